# Initial kernel scaffold; baseline (speedup 1.0000x reference)
import jax
import jax.numpy as jnp
from jax.experimental import pallas as pl


def kernel(x, W_attn, b_attn, W_proj, b_proj):
    raise NotImplementedError("write your pallas kernel here")


# speedup vs baseline: 6.9797x; 6.9797x over previous
import functools

import jax
import jax.numpy as jnp
from jax.experimental import pallas as pl
from jax.experimental.pallas import tpu as pltpu

T = 2048
C = 768
H = 12
HD = C // H
R = 256
NEG = -1e30
SCALE = 1.0 / (HD ** 0.5)
BUDGET = max(1, int(T * 0.2))
KCAP = BUDGET - 1
BIG = 1e30
BIG_BITS = 0x7149F2CA


def _qkv_kernel(x_ref, w_ref, b_ref, o_ref):
    o_ref[...] = (
        jnp.dot(x_ref[...], w_ref[...], preferred_element_type=jnp.float32)
        + b_ref[...]
    )


def _attn_kernel(q_ref, k_ref, v_ref, wp_ref, bp_ref, o_ref, acc_ref):
    pid = pl.program_id(0)

    @pl.when(pid == 0)
    def _init():
        acc_ref[...] = jnp.zeros_like(acc_ref)

    i0 = pid * R
    rows = i0 + jax.lax.broadcasted_iota(jnp.int32, (R, 1), 0)
    cols = jax.lax.broadcasted_iota(jnp.int32, (R, T), 1)

    q0 = q_ref[:, 0:HD]
    k0 = k_ref[:, 0:HD]
    att0 = jax.lax.dot_general(
        q0, k0, (((1,), (1,)), ((), ())), preferred_element_type=jnp.float32
    ) * SCALE
    s_mask = (cols < rows) & (cols > 0)
    S = jnp.where(s_mask, jnp.maximum(att0, 0.0), 0.0)

    ri = jax.lax.broadcasted_iota(jnp.int32, (R, R), 0)
    rj = jax.lax.broadcasted_iota(jnp.int32, (R, R), 1)
    ltri = (rj < ri).astype(jnp.float32)
    local_pref = jnp.dot(ltri, S, preferred_element_type=jnp.float32)
    FF = acc_ref[0:1, :] + local_pref
    acc_ref[0:1, :] = acc_ref[0:1, :] + local_pref[R - 1 : R, :] + S[R - 1 : R, :]

    valid = cols <= rows
    FF_mod = jnp.where(valid, FF, BIG)
    bits = jax.lax.bitcast_convert_type(FF_mod, jnp.int32)
    kk = jnp.minimum(KCAP, rows + 1)

    def bs_body(_, carry):
        lo, hi = carry
        mid = lo + ((hi - lo) >> 1)
        cnt = jnp.sum((bits <= mid).astype(jnp.int32), axis=1, keepdims=True)
        ge = cnt >= kk
        return jnp.where(ge, lo, mid + 1), jnp.where(ge, mid, hi)

    lo0 = jnp.zeros((R, 1), jnp.int32)
    hi0 = jnp.full((R, 1), BIG_BITS + 1, jnp.int32)
    lo, hi = jax.lax.fori_loop(0, 31, bs_body, (lo0, hi0))
    t = lo

    less = bits < t
    nless = jnp.sum(less.astype(jnp.int32), axis=1, keepdims=True)
    eq = bits == t
    pc = eq.astype(jnp.int32)
    shift = 1
    while shift < T:
        rolled = jnp.roll(pc, shift, axis=1)
        pc = pc + jnp.where(cols >= shift, rolled, 0)
        shift *= 2
    keep = less | (eq & (pc <= (kk - nless))) | (cols == rows)

    ys = []
    for h in range(H):
        qh = q_ref[:, h * HD : (h + 1) * HD]
        kh = k_ref[:, h * HD : (h + 1) * HD]
        sh = jax.lax.dot_general(
            qh, kh, (((1,), (1,)), ((), ())), preferred_element_type=jnp.float32
        ) * SCALE
        a = jnp.where(keep, sh - FF, NEG)
        m = jnp.max(a, axis=1, keepdims=True)
        e = jnp.exp(a - m)
        w = e / jnp.sum(e, axis=1, keepdims=True)
        vh = v_ref[:, h * HD : (h + 1) * HD]
        ys.append(jnp.dot(w, vh, preferred_element_type=jnp.float32))
    y = jnp.concatenate(ys, axis=1)

    o_ref[...] = (
        jnp.dot(y, wp_ref[...], preferred_element_type=jnp.float32) + bp_ref[...]
    )


@jax.jit
def kernel(x, W_attn, b_attn, W_proj, b_proj):
    x2 = x.reshape(T, C)

    qkv = pl.pallas_call(
        _qkv_kernel,
        grid=(T // R,),
        in_specs=[
            pl.BlockSpec((R, C), lambda i: (i, 0)),
            pl.BlockSpec((C, 3 * C), lambda i: (0, 0)),
            pl.BlockSpec((1, 3 * C), lambda i: (0, 0)),
        ],
        out_specs=pl.BlockSpec((R, 3 * C), lambda i: (i, 0)),
        out_shape=jax.ShapeDtypeStruct((T, 3 * C), jnp.float32),
    )(x2, W_attn.T, b_attn.reshape(1, 3 * C))

    q, k, v = jnp.split(qkv, 3, axis=1)

    y = pl.pallas_call(
        _attn_kernel,
        grid=(T // R,),
        in_specs=[
            pl.BlockSpec((R, C), lambda i: (i, 0)),
            pl.BlockSpec((T, C), lambda i: (0, 0)),
            pl.BlockSpec((T, C), lambda i: (0, 0)),
            pl.BlockSpec((C, C), lambda i: (0, 0)),
            pl.BlockSpec((1, C), lambda i: (0, 0)),
        ],
        out_specs=pl.BlockSpec((R, C), lambda i: (i, 0)),
        out_shape=jax.ShapeDtypeStruct((T, C), jnp.float32),
        scratch_shapes=[pltpu.VMEM((8, T), jnp.float32)],
        compiler_params=pltpu.CompilerParams(
            dimension_semantics=("arbitrary",),
        ),
    )(q, k, v, W_proj.T, b_proj.reshape(1, C))

    return y.reshape(1, T, C)
